# baseline (speedup 1.0000x reference)
import functools

import jax
import jax.numpy as jnp
from jax import lax
from jax.experimental import pallas as pl
from jax.experimental.pallas import tpu as pltpu
from jax.experimental.pallas import tpu_sc as plsc

_N_LAYERS = 3
_BN_EPS = 1e-5
_N_GRAPHS = 64

_SC_CORES = 2
_SC_SUBCORES = 16
_SCATTER_WIN = 128
_EDGE_BLK = 4000
_N_SLICES = 5


def _f32_to_f16bits(x):
    u = lax.bitcast_convert_type(
        jnp.clip(x, jnp.float32(-60000.0), jnp.float32(60000.0)), jnp.uint32
    )
    mag32 = u & jnp.uint32(0x7FFFFFFF)
    r = mag32 + jnp.uint32(0x0FFF) + ((u >> 13) & jnp.uint32(1))
    mag = r >> 13
    h = jnp.where(
        mag > jnp.uint32(112 << 10), mag - jnp.uint32(112 << 10), jnp.uint32(0)
    )
    return ((u >> 16) & jnp.uint32(0x8000)) | h


def _f16bits_to_f32(h):
    mag = h & jnp.uint32(0x7FFF)
    f = (mag << 13) + jnp.uint32(112 << 23)
    f = jnp.where(mag >= jnp.uint32(1 << 10), f, jnp.uint32(0))
    return lax.bitcast_convert_type(((h & jnp.uint32(0x8000)) << 16) | f, jnp.float32)


def _pack2f16(fv, sv):
    return lax.bitcast_convert_type(
        _f32_to_f16bits(fv) | (_f32_to_f16bits(sv) << 16), jnp.int32
    )


def _unpack2f16(w):
    u = lax.bitcast_convert_type(w, jnp.uint32)
    return _f16bits_to_f32(u & jnp.uint32(0xFFFF)), _f16bits_to_f32(u >> 16)


def _node_tables(h, w_dst, w_src, bias):
    n, f = h.shape

    def body(h_ref, wd_ref, ws_ref, b_ref, pd_ref, ps_ref):
        hh = h_ref[...]
        md = (
            jnp.dot(hh, wd_ref[...], preferred_element_type=jnp.float32)
            + b_ref[...]
        )
        ms = jnp.dot(hh, ws_ref[...], preferred_element_type=jnp.float32)
        pd_ref[...] = _pack2f16(md[:, :f], md[:, f:])
        ps_ref[...] = _pack2f16(ms[:, :f], ms[:, f:])

    return pl.pallas_call(
        body,
        out_shape=[
            jax.ShapeDtypeStruct((n, f), jnp.int32),
            jax.ShapeDtypeStruct((n, f), jnp.int32),
        ],
    )(h, w_dst, w_src, bias)


def _sc_gather(pd, ps, dst2d, src2d):
    n, f = pd.shape
    e = dst2d.shape[1]
    w = 128
    mesh = plsc.VectorSubcoreMesh(core_axis_name="c", subcore_axis_name="s")

    @functools.partial(
        pl.kernel,
        out_type=[
            jax.ShapeDtypeStruct((e, f), jnp.int32),
            jax.ShapeDtypeStruct((e, f), jnp.int32),
        ],
        mesh=mesh,
        scratch_types=[pltpu.SemaphoreType.DMA, pltpu.SemaphoreType.DMA],
    )
    def k(pd_hbm, ps_hbm, di_hbm, si_hbm, gd_hbm, gs_hbm, sem0, sem1):
        def body(di_v, si_v, gd_v, gs_v):
            cp0 = pltpu.async_copy(pd_hbm.at[di_v.at[0]], gd_v, sem0)
            cp1 = pltpu.async_copy(ps_hbm.at[si_v.at[0]], gs_v, sem1)
            cp0.wait()
            cp1.wait()

        pltpu.emit_pipeline(
            body,
            grid=(e // w,),
            in_specs=[
                pl.BlockSpec((1, w), lambda i: (0, i)),
                pl.BlockSpec((1, w), lambda i: (0, i)),
            ],
            out_specs=[
                pl.BlockSpec((w, f), lambda i: (i, 0)),
                pl.BlockSpec((w, f), lambda i: (i, 0)),
            ],
            core_axis_name=("c", "s"),
            dimension_semantics=(pltpu.PARALLEL,),
        )(di_hbm, si_hbm, gd_hbm, gs_hbm)

    return k(pd, ps, dst2d, src2d)


def _edge_compute(gd, gs, edge_attr, w_edge):
    e, f = gd.shape
    c = w_edge.shape[1]
    d = edge_attr.shape[1]
    blk = _EDGE_BLK

    def body(gd_ref, gs_ref, ea_ref, we_ref, msg_ref):
        ce = jnp.dot(ea_ref[...], we_ref[...], preferred_element_type=jnp.float32)
        df, ds = _unpack2f16(gd_ref[...])
        sf, ss = _unpack2f16(gs_ref[...])
        zf = df + sf + ce[:, :f]
        zs = ds + ss + ce[:, f:]
        sp = jnp.maximum(zs, 0.0) + jnp.log1p(jnp.exp(-jnp.abs(zs)))
        msg_ref[...] = jax.nn.sigmoid(zf) * sp

    return pl.pallas_call(
        body,
        grid=(e // blk,),
        in_specs=[
            pl.BlockSpec((blk, f), lambda i: (i, 0)),
            pl.BlockSpec((blk, f), lambda i: (i, 0)),
            pl.BlockSpec((blk, d), lambda i: (i, 0)),
            pl.BlockSpec((d, c), lambda i: (0, 0)),
        ],
        out_specs=pl.BlockSpec((blk, f), lambda i: (i, 0)),
        out_shape=jax.ShapeDtypeStruct((e, f), jnp.float32),
    )(gd, gs, edge_attr, w_edge)


def _sc_scatter(msgs, dst3ds, zeros_nf):
    f = msgs[0].shape[1]
    n = zeros_nf.shape[0]
    w = _SCATTER_WIN
    sub = w // 128
    rows = n // _SC_SUBCORES
    nsl = len(msgs)
    mesh = plsc.VectorSubcoreMesh(core_axis_name="c", subcore_axis_name="s")

    @functools.partial(
        pl.kernel,
        out_type=jax.ShapeDtypeStruct((_SC_CORES, n, f), jnp.float32),
        mesh=mesh,
        scratch_types=[pltpu.VMEM_SHARED((n, f), jnp.float32)],
    )
    def k(*refs):
        msg_hbms = refs[:nsl]
        di_hbms = refs[nsl : 2 * nsl]
        z_hbm = refs[2 * nsl]
        out_hbm = refs[2 * nsl + 1]
        acc = refs[2 * nsl + 2]
        sid = lax.axis_index("s")
        cid = lax.axis_index("c")
        pltpu.sync_copy(
            z_hbm.at[pl.ds(sid * rows, rows)], acc.at[pl.ds(sid * rows, rows)]
        )
        plsc.subcore_barrier()

        def body(i_v, m_v):
            for j in range(sub):
                pltpu.sync_copy(
                    m_v.at[pl.ds(j * 128, 128)], acc.at[i_v.at[0, j]], add=True
                )

        for s in range(nsl):
            e_s = msgs[s].shape[0]
            pltpu.emit_pipeline(
                body,
                grid=(e_s // w,),
                in_specs=[
                    pl.BlockSpec((1, sub, 128), lambda i: (i, 0, 0)),
                    pl.BlockSpec((w, f), lambda i: (i, 0)),
                ],
                out_specs=[],
                core_axis_name=("c", "s"),
                dimension_semantics=(pltpu.PARALLEL,),
            )(di_hbms[s], msg_hbms[s])

        plsc.subcore_barrier()
        pltpu.sync_copy(
            acc.at[pl.ds(sid * rows, rows)],
            out_hbm.at[cid, pl.ds(sid * rows, rows)],
        )

    return k(*msgs, *dst3ds, zeros_nf)


def _bn_residual(parts_list, h, gamma, beta):
    n, f = h.shape

    def body(*refs):
        p_refs = refs[: len(parts_list)]
        h_ref, g_ref, b_ref, o_ref = refs[len(parts_list) :]
        acc = p_refs[0][0] + p_refs[0][1]
        for pr in p_refs[1:]:
            acc = acc + pr[0] + pr[1]
        scale = g_ref[...] * (1.0 / jnp.sqrt(1.0 + _BN_EPS))
        o_ref[...] = acc * scale + b_ref[...] + h_ref[...]

    return pl.pallas_call(
        body,
        grid=(1,),
        in_specs=[
            pl.BlockSpec((2, n, f), lambda i: (0, 0, 0))
            for _ in parts_list
        ]
        + [
            pl.BlockSpec((n, f), lambda i: (0, 0)),
            pl.BlockSpec((1, f), lambda i: (0, 0)),
            pl.BlockSpec((1, f), lambda i: (0, 0)),
        ],
        out_specs=pl.BlockSpec((n, f), lambda i: (0, 0)),
        out_shape=jax.ShapeDtypeStruct((n, f), jnp.float32),
    )(*parts_list, h, gamma, beta)


def _pool_head(h, batch2d, lin_w, lin_b):
    n, f = h.shape

    def body(h_ref, b_ref, w_ref, bb_ref, o_ref):
        bt = b_ref[...]
        onehot_t = (
            lax.broadcasted_iota(jnp.int32, (_N_GRAPHS, n), 0) == bt
        ).astype(jnp.float32)
        sums = jnp.dot(onehot_t, h_ref[...], preferred_element_type=jnp.float32)
        cnt = jnp.sum(onehot_t, axis=1, keepdims=True)
        mean = sums / jnp.maximum(cnt, 1.0)
        o_ref[...] = (
            jnp.dot(mean, w_ref[...], preferred_element_type=jnp.float32)
            + bb_ref[...]
        )

    return pl.pallas_call(
        body,
        out_shape=jax.ShapeDtypeStruct((_N_GRAPHS, 1), jnp.float32),
    )(h, batch2d, lin_w, lin_b)


def kernel(x, edge_index, edge_attr, batch, Wf, bf, Ws, bs, gamma, beta, linW, linb):
    n, f = x.shape
    e = edge_index.shape[1]
    n_pad = ((n + 8 * _SC_SUBCORES - 1) // (8 * _SC_SUBCORES)) * 8 * _SC_SUBCORES
    zeros_nf = jnp.zeros((n_pad, f), jnp.float32)

    es = e // _N_SLICES
    src_sl = [edge_index[0, s * es : (s + 1) * es].reshape(1, es) for s in range(_N_SLICES)]
    dst_sl = [edge_index[1, s * es : (s + 1) * es].reshape(1, es) for s in range(_N_SLICES)]
    dst3d_sl = [
        edge_index[1, s * es : (s + 1) * es].reshape(
            es // _SCATTER_WIN, _SCATTER_WIN // 128, 128
        )
        for s in range(_N_SLICES)
    ]
    ea_sl = [edge_attr[s * es : (s + 1) * es] for s in range(_N_SLICES)]

    h = x
    for l in range(_N_LAYERS):
        w_dst = jnp.concatenate([Wf[l, :f], Ws[l, :f]], axis=1)
        w_src = jnp.concatenate([Wf[l, f : 2 * f], Ws[l, f : 2 * f]], axis=1)
        w_edge = jnp.concatenate([Wf[l, 2 * f :], Ws[l, 2 * f :]], axis=1)
        bias = jnp.concatenate([bf[l], bs[l]]).reshape(1, 2 * f)
        pd, ps = _node_tables(h, w_dst, w_src, bias)
        parts_list = []
        for s in range(_N_SLICES):
            gdv, gsv = _sc_gather(pd, ps, dst_sl[s], src_sl[s])
            msg = _edge_compute(gdv, gsv, ea_sl[s], w_edge)
            parts_list.append(_sc_scatter([msg], [dst3d_sl[s]], zeros_nf))
        h = _bn_residual(parts_list, h, gamma[l].reshape(1, f), beta[l].reshape(1, f))

    return _pool_head(h, batch.reshape(1, n), linW, linb.reshape(1, 1))


# speedup vs baseline: 1.0110x; 1.0110x over previous
import functools

import jax
import jax.numpy as jnp
from jax import lax
from jax.experimental import pallas as pl
from jax.experimental.pallas import tpu as pltpu
from jax.experimental.pallas import tpu_sc as plsc

_N_LAYERS = 3
_BN_EPS = 1e-5
_N_GRAPHS = 64

_SC_CORES = 2
_SC_SUBCORES = 16
_SCATTER_WIN = 128
_EDGE_BLK = 4000
_N_SLICES = 4


def _f32_to_f16bits(x):
    u = lax.bitcast_convert_type(
        jnp.clip(x, jnp.float32(-60000.0), jnp.float32(60000.0)), jnp.uint32
    )
    mag32 = u & jnp.uint32(0x7FFFFFFF)
    r = mag32 + jnp.uint32(0x0FFF) + ((u >> 13) & jnp.uint32(1))
    mag = r >> 13
    h = jnp.where(
        mag > jnp.uint32(112 << 10), mag - jnp.uint32(112 << 10), jnp.uint32(0)
    )
    return ((u >> 16) & jnp.uint32(0x8000)) | h


def _f16bits_to_f32(h):
    mag = h & jnp.uint32(0x7FFF)
    f = (mag << 13) + jnp.uint32(112 << 23)
    f = jnp.where(mag >= jnp.uint32(1 << 10), f, jnp.uint32(0))
    return lax.bitcast_convert_type(((h & jnp.uint32(0x8000)) << 16) | f, jnp.float32)


def _pack2f16(fv, sv):
    return lax.bitcast_convert_type(
        _f32_to_f16bits(fv) | (_f32_to_f16bits(sv) << 16), jnp.int32
    )


def _unpack2f16(w):
    u = lax.bitcast_convert_type(w, jnp.uint32)
    return _f16bits_to_f32(u & jnp.uint32(0xFFFF)), _f16bits_to_f32(u >> 16)


def _node_tables(h, w_dst, w_src, bias):
    n, f = h.shape

    def body(h_ref, wd_ref, ws_ref, b_ref, pd_ref, ps_ref):
        hh = h_ref[...]
        md = (
            jnp.dot(hh, wd_ref[...], preferred_element_type=jnp.float32)
            + b_ref[...]
        )
        ms = jnp.dot(hh, ws_ref[...], preferred_element_type=jnp.float32)
        pd_ref[...] = _pack2f16(md[:, :f], md[:, f:])
        ps_ref[...] = _pack2f16(ms[:, :f], ms[:, f:])

    return pl.pallas_call(
        body,
        out_shape=[
            jax.ShapeDtypeStruct((n, f), jnp.int32),
            jax.ShapeDtypeStruct((n, f), jnp.int32),
        ],
    )(h, w_dst, w_src, bias)


def _sc_gather(pd, ps, dst2d, src2d):
    n, f = pd.shape
    e = dst2d.shape[1]
    w = 128
    mesh = plsc.VectorSubcoreMesh(core_axis_name="c", subcore_axis_name="s")

    @functools.partial(
        pl.kernel,
        out_type=[
            jax.ShapeDtypeStruct((e, f), jnp.int32),
            jax.ShapeDtypeStruct((e, f), jnp.int32),
        ],
        mesh=mesh,
        scratch_types=[pltpu.SemaphoreType.DMA, pltpu.SemaphoreType.DMA],
    )
    def k(pd_hbm, ps_hbm, di_hbm, si_hbm, gd_hbm, gs_hbm, sem0, sem1):
        def body(di_v, si_v, gd_v, gs_v):
            cp0 = pltpu.async_copy(pd_hbm.at[di_v.at[0]], gd_v, sem0)
            cp1 = pltpu.async_copy(ps_hbm.at[si_v.at[0]], gs_v, sem1)
            cp0.wait()
            cp1.wait()

        pltpu.emit_pipeline(
            body,
            grid=(e // w,),
            in_specs=[
                pl.BlockSpec((1, w), lambda i: (0, i)),
                pl.BlockSpec((1, w), lambda i: (0, i)),
            ],
            out_specs=[
                pl.BlockSpec((w, f), lambda i: (i, 0)),
                pl.BlockSpec((w, f), lambda i: (i, 0)),
            ],
            core_axis_name=("c", "s"),
            dimension_semantics=(pltpu.PARALLEL,),
        )(di_hbm, si_hbm, gd_hbm, gs_hbm)

    return k(pd, ps, dst2d, src2d)


def _edge_compute(gd, gs, edge_attr, w_edge):
    e, f = gd.shape
    c = w_edge.shape[1]
    d = edge_attr.shape[1]
    blk = _EDGE_BLK

    def body(gd_ref, gs_ref, ea_ref, we_ref, msg_ref):
        ce = jnp.dot(ea_ref[...], we_ref[...], preferred_element_type=jnp.float32)
        df, ds = _unpack2f16(gd_ref[...])
        sf, ss = _unpack2f16(gs_ref[...])
        zf = df + sf + ce[:, :f]
        zs = ds + ss + ce[:, f:]
        sp = jnp.maximum(zs, 0.0) + jnp.log1p(jnp.exp(-jnp.abs(zs)))
        msg_ref[...] = jax.nn.sigmoid(zf) * sp

    return pl.pallas_call(
        body,
        grid=(e // blk,),
        in_specs=[
            pl.BlockSpec((blk, f), lambda i: (i, 0)),
            pl.BlockSpec((blk, f), lambda i: (i, 0)),
            pl.BlockSpec((blk, d), lambda i: (i, 0)),
            pl.BlockSpec((d, c), lambda i: (0, 0)),
        ],
        out_specs=pl.BlockSpec((blk, f), lambda i: (i, 0)),
        out_shape=jax.ShapeDtypeStruct((e, f), jnp.float32),
    )(gd, gs, edge_attr, w_edge)


def _sc_scatter(msgs, dst3ds, zeros_nf):
    f = msgs[0].shape[1]
    n = zeros_nf.shape[0]
    w = _SCATTER_WIN
    sub = w // 128
    rows = n // _SC_SUBCORES
    nsl = len(msgs)
    mesh = plsc.VectorSubcoreMesh(core_axis_name="c", subcore_axis_name="s")

    @functools.partial(
        pl.kernel,
        out_type=jax.ShapeDtypeStruct((_SC_CORES, n, f), jnp.float32),
        mesh=mesh,
        scratch_types=[pltpu.VMEM_SHARED((n, f), jnp.float32)],
    )
    def k(*refs):
        msg_hbms = refs[:nsl]
        di_hbms = refs[nsl : 2 * nsl]
        z_hbm = refs[2 * nsl]
        out_hbm = refs[2 * nsl + 1]
        acc = refs[2 * nsl + 2]
        sid = lax.axis_index("s")
        cid = lax.axis_index("c")
        pltpu.sync_copy(
            z_hbm.at[pl.ds(sid * rows, rows)], acc.at[pl.ds(sid * rows, rows)]
        )
        plsc.subcore_barrier()

        def body(i_v, m_v):
            for j in range(sub):
                pltpu.sync_copy(
                    m_v.at[pl.ds(j * 128, 128)], acc.at[i_v.at[0, j]], add=True
                )

        for s in range(nsl):
            e_s = msgs[s].shape[0]
            pltpu.emit_pipeline(
                body,
                grid=(e_s // w,),
                in_specs=[
                    pl.BlockSpec((1, sub, 128), lambda i: (i, 0, 0)),
                    pl.BlockSpec((w, f), lambda i: (i, 0)),
                ],
                out_specs=[],
                core_axis_name=("c", "s"),
                dimension_semantics=(pltpu.PARALLEL,),
            )(di_hbms[s], msg_hbms[s])

        plsc.subcore_barrier()
        pltpu.sync_copy(
            acc.at[pl.ds(sid * rows, rows)],
            out_hbm.at[cid, pl.ds(sid * rows, rows)],
        )

    return k(*msgs, *dst3ds, zeros_nf)


def _bn_residual(parts_list, h, gamma, beta):
    n, f = h.shape
    blk = 2000

    def body(*refs):
        p_refs = refs[: len(parts_list)]
        h_ref, g_ref, b_ref, o_ref = refs[len(parts_list) :]
        acc = p_refs[0][0] + p_refs[0][1]
        for pr in p_refs[1:]:
            acc = acc + pr[0] + pr[1]
        scale = g_ref[...] * (1.0 / jnp.sqrt(1.0 + _BN_EPS))
        o_ref[...] = acc * scale + b_ref[...] + h_ref[...]

    return pl.pallas_call(
        body,
        grid=(n // blk,),
        in_specs=[
            pl.BlockSpec((2, blk, f), lambda i: (0, i, 0))
            for _ in parts_list
        ]
        + [
            pl.BlockSpec((blk, f), lambda i: (i, 0)),
            pl.BlockSpec((1, f), lambda i: (0, 0)),
            pl.BlockSpec((1, f), lambda i: (0, 0)),
        ],
        out_specs=pl.BlockSpec((blk, f), lambda i: (i, 0)),
        out_shape=jax.ShapeDtypeStruct((n, f), jnp.float32),
    )(*parts_list, h, gamma, beta)


def _pool_head(h, batch2d, lin_w, lin_b):
    n, f = h.shape

    def body(h_ref, b_ref, w_ref, bb_ref, o_ref):
        bt = b_ref[...]
        onehot_t = (
            lax.broadcasted_iota(jnp.int32, (_N_GRAPHS, n), 0) == bt
        ).astype(jnp.float32)
        sums = jnp.dot(onehot_t, h_ref[...], preferred_element_type=jnp.float32)
        cnt = jnp.sum(onehot_t, axis=1, keepdims=True)
        mean = sums / jnp.maximum(cnt, 1.0)
        o_ref[...] = (
            jnp.dot(mean, w_ref[...], preferred_element_type=jnp.float32)
            + bb_ref[...]
        )

    return pl.pallas_call(
        body,
        out_shape=jax.ShapeDtypeStruct((_N_GRAPHS, 1), jnp.float32),
    )(h, batch2d, lin_w, lin_b)


def kernel(x, edge_index, edge_attr, batch, Wf, bf, Ws, bs, gamma, beta, linW, linb):
    n, f = x.shape
    e = edge_index.shape[1]
    n_pad = ((n + 8 * _SC_SUBCORES - 1) // (8 * _SC_SUBCORES)) * 8 * _SC_SUBCORES
    zeros_nf = jnp.zeros((n_pad, f), jnp.float32)

    es = e // _N_SLICES
    src_sl = [edge_index[0, s * es : (s + 1) * es].reshape(1, es) for s in range(_N_SLICES)]
    dst_sl = [edge_index[1, s * es : (s + 1) * es].reshape(1, es) for s in range(_N_SLICES)]
    dst3d_sl = [
        edge_index[1, s * es : (s + 1) * es].reshape(
            es // _SCATTER_WIN, _SCATTER_WIN // 128, 128
        )
        for s in range(_N_SLICES)
    ]
    ea_sl = [edge_attr[s * es : (s + 1) * es] for s in range(_N_SLICES)]

    h = x
    for l in range(_N_LAYERS):
        w_dst = jnp.concatenate([Wf[l, :f], Ws[l, :f]], axis=1)
        w_src = jnp.concatenate([Wf[l, f : 2 * f], Ws[l, f : 2 * f]], axis=1)
        w_edge = jnp.concatenate([Wf[l, 2 * f :], Ws[l, 2 * f :]], axis=1)
        bias = jnp.concatenate([bf[l], bs[l]]).reshape(1, 2 * f)
        pd, ps = _node_tables(h, w_dst, w_src, bias)
        parts_list = []
        for s in range(_N_SLICES):
            gdv, gsv = _sc_gather(pd, ps, dst_sl[s], src_sl[s])
            msg = _edge_compute(gdv, gsv, ea_sl[s], w_edge)
            parts_list.append(_sc_scatter([msg], [dst3d_sl[s]], zeros_nf))
        h = _bn_residual(parts_list, h, gamma[l].reshape(1, f), beta[l].reshape(1, f))

    return _pool_head(h, batch.reshape(1, n), linW, linb.reshape(1, 1))
